# baseline (speedup 1.0000x reference)
import jax
import jax.numpy as jnp
from jax import lax
from jax.experimental import pallas as pl
from jax.experimental.pallas import tpu as pltpu
from jax.experimental.pallas import tpu_sc as plsc

_K = 8192
_C = 256
_B = 8
_H = 24
_W = 24
_HW = _H * _W
_N = _B * _HW
_KCHUNK = 1024
_BETA = 0.25

_NW = 32
_TPW = _N // _NW
_HALF = _TPW // 2


def _argmin_body(x_ref, w_ref, idx_ref, loss_ref, acc_ref):
    x_cb = x_ref[0].reshape(_C, _HW)
    xm2 = x_cb * (-2.0)
    x2 = jnp.sum(x_cb * x_cb, axis=0)
    iota_f = lax.broadcasted_iota(jnp.int32, (_KCHUNK, _HW), 0).astype(jnp.float32)
    best_v = jnp.full((_HW,), jnp.inf, dtype=jnp.float32)
    best_i = jnp.zeros((_HW,), dtype=jnp.float32)
    for k in range(_K // _KCHUNK):
        w = w_ref[pl.ds(k * _KCHUNK, _KCHUNK), :]
        w2 = jnp.sum(w * w, axis=1)
        mm2 = lax.dot_general(w, xm2, (((1,), (0,)), ((), ())))
        d2 = (x2[None, :] + w2[:, None]) + mm2
        bv = jnp.min(d2, axis=0)
        bi = jnp.min(jnp.where(d2 == bv[None, :], iota_f, 65536.0), axis=0)
        upd = bv < best_v
        best_v = jnp.where(upd, bv, best_v)
        best_i = jnp.where(upd, bi + jnp.float32(k * _KCHUNK), best_i)
    idx_ref[0, 0] = best_i.astype(jnp.int32)
    b = pl.program_id(0)
    prev = jnp.where(b == 0, 0.0, acc_ref[0])
    acc_ref[0] = prev + jnp.sum(best_v)

    @pl.when(b == _B - 1)
    def _tail():
        loss = (1.0 + _BETA) * acc_ref[0] / jnp.float32(_N * _C)
        loss_ref[...] = jnp.full((1, 1), loss, dtype=jnp.float32)


def _tc_argmin(x_cf, emb):
    return pl.pallas_call(
        _argmin_body,
        grid=(_B,),
        in_specs=[
            pl.BlockSpec((1, _C, _H, _W), lambda b: (b, 0, 0, 0)),
            pl.BlockSpec((_K, _C), lambda b: (0, 0)),
        ],
        out_specs=[
            pl.BlockSpec((1, 1, _HW), lambda b: (b, 0, 0)),
            pl.BlockSpec((1, 1), lambda b: (0, 0)),
        ],
        out_shape=[
            jax.ShapeDtypeStruct((_B, 1, _HW), jnp.int32),
            jax.ShapeDtypeStruct((1, 1), jnp.float32),
        ],
        scratch_shapes=[pltpu.SMEM((1,), jnp.float32)],
    )(x_cf, emb)


def _sc_body(idx_hbm, emb_hbm, zeros_hbm, xq_hbm, counts_hbm,
             idx_v, rows_v, ones_v, hist_sh, sem):
    c = lax.axis_index("c")
    s = lax.axis_index("s")
    wid = s * 2 + c
    base = wid * _TPW
    pltpu.sync_copy(idx_hbm.at[pl.ds(base, _HALF)], idx_v.at[0])
    pltpu.sync_copy(idx_hbm.at[pl.ds(base + _HALF, _HALF)], idx_v.at[1])
    cp0 = pltpu.async_copy(emb_hbm.at[idx_v.at[0]], rows_v.at[pl.ds(0, _HALF)], sem)
    cp1 = pltpu.async_copy(emb_hbm.at[idx_v.at[1]], rows_v.at[pl.ds(_HALF, _HALF)], sem)
    for i in range(_TPW // 16):
        ones_v[pl.ds(i * 16, 16)] = jnp.full((16,), 1.0, dtype=jnp.float32)

    @pl.when(s == 0)
    def _zero_hist():
        pltpu.sync_copy(zeros_hbm, hist_sh)

    plsc.subcore_barrier()
    pltpu.sync_copy(ones_v.at[pl.ds(0, _HALF)], hist_sh.at[idx_v.at[0]], add=True)
    pltpu.sync_copy(ones_v.at[pl.ds(_HALF, _HALF)], hist_sh.at[idx_v.at[1]], add=True)
    plsc.subcore_barrier()

    @pl.when(s == 0)
    def _write_counts():
        pltpu.sync_copy(hist_sh, counts_hbm.at[c])

    cp0.wait()
    cp1.wait()
    pltpu.sync_copy(rows_v, xq_hbm.at[pl.ds(base, _TPW)])


def _sc_gather_hist(idxs, emb, zeros):
    fn = pl.kernel(
        _sc_body,
        out_type=[
            jax.ShapeDtypeStruct((_N, _C), jnp.float32),
            jax.ShapeDtypeStruct((2, _K), jnp.float32),
        ],
        mesh=plsc.VectorSubcoreMesh(core_axis_name="c", subcore_axis_name="s"),
        scratch_types=[
            pltpu.VMEM((2, _HALF), jnp.int32),
            pltpu.VMEM((_TPW, _C), jnp.float32),
            pltpu.VMEM((_TPW,), jnp.float32),
            pltpu.VMEM_SHARED((_K,), jnp.float32),
            pltpu.SemaphoreType.DMA,
        ],
    )
    return fn(idxs, emb, zeros)


def _finish_body(xq_ref, cnt_ref, out_ref, perp_ref):
    b = pl.program_id(0)
    out_ref[0] = xq_ref[0].T.reshape(_C, _H, _W)

    @pl.when(b == _B - 1)
    def _tail():
        cnt = cnt_ref[0:1, :] + cnt_ref[1:2, :]
        p = cnt / jnp.float32(_N)
        ent = -jnp.sum(p * jnp.log(p + 1e-10))
        perp_ref[...] = jnp.full((1, 1), jnp.exp(ent), dtype=jnp.float32)


def _tc_finish(xq3, counts2):
    return pl.pallas_call(
        _finish_body,
        grid=(_B,),
        in_specs=[
            pl.BlockSpec((1, _HW, _C), lambda b: (b, 0, 0)),
            pl.BlockSpec((2, _K), lambda b: (0, 0)),
        ],
        out_specs=[
            pl.BlockSpec((1, _C, _H, _W), lambda b: (b, 0, 0, 0)),
            pl.BlockSpec((1, 1), lambda b: (0, 0)),
        ],
        out_shape=[
            jax.ShapeDtypeStruct((_B, _C, _H, _W), jnp.float32),
            jax.ShapeDtypeStruct((1, 1), jnp.float32),
        ],
    )(xq3, counts2)


def kernel(x_in, emb_weight):
    idxs3, loss = _tc_argmin(x_in, emb_weight)
    idxs = idxs3.reshape(_N)
    zeros = jnp.zeros((_K,), jnp.float32)
    xq_flat, counts2 = _sc_gather_hist(idxs, emb_weight, zeros)
    xq3 = xq_flat.reshape(_B, _HW, _C)
    xq_out, perp = _tc_finish(xq3, counts2)
    return (
        xq_out,
        idxs.reshape(_B, _H, _W),
        loss[0, 0],
        perp[0, 0],
    )


# speedup vs baseline: 1.6896x; 1.6896x over previous
import jax
import jax.numpy as jnp
from jax import lax
from jax.experimental import pallas as pl
from jax.experimental.pallas import tpu as pltpu
from jax.experimental.pallas import tpu_sc as plsc

_K = 8192
_C = 256
_B = 8
_H = 24
_W = 24
_HW = _H * _W
_N = _B * _HW
_KCHUNK = 1024
_BETA = 0.25

_NW = 32
_TPW = _N // _NW
_HALF = _TPW // 2


def _argmin_body(x_ref, w_ref, idx_ref, loss_ref, acc_ref):
    x_cb = x_ref[0]
    xm2 = x_cb * (-2.0)
    x2 = jnp.sum(x_cb * x_cb, axis=0)
    best8 = jnp.full((8, _HW), jnp.inf, dtype=jnp.float32)
    besti8 = jnp.zeros((8, _HW), dtype=jnp.int32)
    for k in range(_K // _KCHUNK):
        w = w_ref[pl.ds(k * _KCHUNK, _KCHUNK), :]
        w2 = jnp.sum(w * w, axis=1)
        mm2 = lax.dot_general(w, xm2, (((1,), (0,)), ((), ())))
        d2 = (x2[None, :] + w2[:, None]) + mm2
        for i in range(_KCHUNK // 8):
            d2s = lax.slice(d2, (i * 8, 0), (i * 8 + 8, _HW))
            m = d2s < best8
            best8 = jnp.where(m, d2s, best8)
            besti8 = jnp.where(m, jnp.int32(k * (_KCHUNK // 8) + i), besti8)
    best_v = jnp.min(best8, axis=0)
    s_iota = lax.broadcasted_iota(jnp.int32, (8, _HW), 0)
    gk = besti8 * 8 + s_iota
    cand = jnp.where(best8 == best_v[None, :], gk, jnp.int32(_K))
    idx_ref[0, 0] = jnp.min(cand, axis=0)
    b = pl.program_id(0)
    prev = jnp.where(b == 0, 0.0, acc_ref[0])
    acc_ref[0] = prev + jnp.sum(best_v)

    @pl.when(b == _B - 1)
    def _tail():
        loss = (1.0 + _BETA) * acc_ref[0] / jnp.float32(_N * _C)
        loss_ref[...] = jnp.full((1, 1), loss, dtype=jnp.float32)


def _tc_argmin(x_cf, emb):
    return pl.pallas_call(
        _argmin_body,
        grid=(_B,),
        in_specs=[
            pl.BlockSpec((1, _C, _HW), lambda b: (b, 0, 0)),
            pl.BlockSpec((_K, _C), lambda b: (0, 0)),
        ],
        out_specs=[
            pl.BlockSpec((1, 1, _HW), lambda b: (b, 0, 0)),
            pl.BlockSpec((1, 1), lambda b: (0, 0)),
        ],
        out_shape=[
            jax.ShapeDtypeStruct((_B, 1, _HW), jnp.int32),
            jax.ShapeDtypeStruct((1, 1), jnp.float32),
        ],
        scratch_shapes=[pltpu.SMEM((1,), jnp.float32)],
    )(x_cf, emb)


def _sc_body(idx_hbm, emb_hbm, zeros_hbm, xq_hbm, counts_hbm,
             idx_v, rows_v, ones_v, hist_sh, sem):
    c = lax.axis_index("c")
    s = lax.axis_index("s")
    wid = s * 2 + c
    base = wid * _TPW
    pltpu.sync_copy(idx_hbm.at[pl.ds(base, _HALF)], idx_v.at[0])
    pltpu.sync_copy(idx_hbm.at[pl.ds(base + _HALF, _HALF)], idx_v.at[1])
    cp0 = pltpu.async_copy(emb_hbm.at[idx_v.at[0]], rows_v.at[pl.ds(0, _HALF)], sem)
    cp1 = pltpu.async_copy(emb_hbm.at[idx_v.at[1]], rows_v.at[pl.ds(_HALF, _HALF)], sem)
    for i in range(_TPW // 16):
        ones_v[pl.ds(i * 16, 16)] = jnp.full((16,), 1.0, dtype=jnp.float32)

    @pl.when(s == 0)
    def _zero_hist():
        pltpu.sync_copy(zeros_hbm, hist_sh)

    plsc.subcore_barrier()
    pltpu.sync_copy(ones_v.at[pl.ds(0, _HALF)], hist_sh.at[idx_v.at[0]], add=True)
    pltpu.sync_copy(ones_v.at[pl.ds(_HALF, _HALF)], hist_sh.at[idx_v.at[1]], add=True)
    plsc.subcore_barrier()

    @pl.when(s == 0)
    def _write_counts():
        pltpu.sync_copy(hist_sh, counts_hbm.at[c])

    cp0.wait()
    cp1.wait()
    pltpu.sync_copy(rows_v, xq_hbm.at[pl.ds(base, _TPW)])


def _sc_gather_hist(idxs, emb, zeros):
    fn = pl.kernel(
        _sc_body,
        out_type=[
            jax.ShapeDtypeStruct((_N, _C), jnp.float32),
            jax.ShapeDtypeStruct((2, _K), jnp.float32),
        ],
        mesh=plsc.VectorSubcoreMesh(core_axis_name="c", subcore_axis_name="s"),
        scratch_types=[
            pltpu.VMEM((2, _HALF), jnp.int32),
            pltpu.VMEM((_TPW, _C), jnp.float32),
            pltpu.VMEM((_TPW,), jnp.float32),
            pltpu.VMEM_SHARED((_K,), jnp.float32),
            pltpu.SemaphoreType.DMA,
        ],
    )
    return fn(idxs, emb, zeros)


def _finish_body(xq_ref, cnt_ref, out_ref, perp_ref):
    b = pl.program_id(0)
    out_ref[0] = xq_ref[0].T

    @pl.when(b == _B - 1)
    def _tail():
        cnt = cnt_ref[0:1, :] + cnt_ref[1:2, :]
        p = cnt / jnp.float32(_N)
        ent = -jnp.sum(p * jnp.log(p + 1e-10))
        perp_ref[...] = jnp.full((1, 1), jnp.exp(ent), dtype=jnp.float32)


def _tc_finish(xq3, counts2):
    return pl.pallas_call(
        _finish_body,
        grid=(_B,),
        in_specs=[
            pl.BlockSpec((1, _HW, _C), lambda b: (b, 0, 0)),
            pl.BlockSpec((2, _K), lambda b: (0, 0)),
        ],
        out_specs=[
            pl.BlockSpec((1, _C, _HW), lambda b: (b, 0, 0)),
            pl.BlockSpec((1, 1), lambda b: (0, 0)),
        ],
        out_shape=[
            jax.ShapeDtypeStruct((_B, _C, _HW), jnp.float32),
            jax.ShapeDtypeStruct((1, 1), jnp.float32),
        ],
    )(xq3, counts2)


def kernel(x_in, emb_weight):
    x_cf = x_in.reshape(_B, _C, _HW)
    idxs3, loss = _tc_argmin(x_cf, emb_weight)
    idxs = idxs3.reshape(_N)
    zeros = jnp.zeros((_K,), jnp.float32)
    xq_flat, counts2 = _sc_gather_hist(idxs, emb_weight, zeros)
    xq3 = xq_flat.reshape(_B, _HW, _C)
    xq_out, perp = _tc_finish(xq3, counts2)
    return (
        xq_out.reshape(_B, _C, _H, _W),
        idxs.reshape(_B, _H, _W),
        loss[0, 0],
        perp[0, 0],
    )


# speedup vs baseline: 1.7104x; 1.0123x over previous
import jax
import jax.numpy as jnp
from jax import lax
from jax.experimental import pallas as pl
from jax.experimental.pallas import tpu as pltpu
from jax.experimental.pallas import tpu_sc as plsc

_K = 8192
_C = 256
_B = 8
_H = 24
_W = 24
_HW = _H * _W
_N = _B * _HW
_KCHUNK = 1024
_BETA = 0.25

_NW = 32
_TPW = _N // _NW
_HALF = _TPW // 2


def _argmin_body(x_ref, w_ref, idx_ref, loss_ref, acc_ref, wm2_ref, w2_ref):
    b = pl.program_id(0)

    @pl.when(b == 0)
    def _prep():
        w = w_ref[...]
        wm2_ref[...] = w * (-2.0)
        w2_ref[...] = jnp.sum(w * w, axis=1, keepdims=True)

    x_cb = x_ref[0]
    x2 = jnp.sum(x_cb * x_cb, axis=0)
    best8 = jnp.full((8, _HW), jnp.inf, dtype=jnp.float32)
    besti8 = jnp.zeros((8, _HW), dtype=jnp.int32)
    for k in range(_K // _KCHUNK):
        wv = wm2_ref[pl.ds(k * _KCHUNK, _KCHUNK), :]
        w2c = w2_ref[pl.ds(k * _KCHUNK, _KCHUNK), :]
        mm2 = lax.dot_general(wv, x_cb, (((1,), (0,)), ((), ())))
        d2 = (x2[None, :] + w2c) + mm2
        for i in range(_KCHUNK // 8):
            d2s = lax.slice(d2, (i * 8, 0), (i * 8 + 8, _HW))
            m = d2s < best8
            best8 = jnp.where(m, d2s, best8)
            besti8 = jnp.where(m, jnp.int32(k * (_KCHUNK // 8) + i), besti8)
    best_v = jnp.min(best8, axis=0)
    s_iota = lax.broadcasted_iota(jnp.int32, (8, _HW), 0)
    gk = besti8 * 8 + s_iota
    cand = jnp.where(best8 == best_v[None, :], gk, jnp.int32(_K))
    idx_ref[0, 0] = jnp.min(cand, axis=0)
    prev = jnp.where(b == 0, 0.0, acc_ref[0])
    acc_ref[0] = prev + jnp.sum(best_v)

    @pl.when(b == _B - 1)
    def _tail():
        loss = (1.0 + _BETA) * acc_ref[0] / jnp.float32(_N * _C)
        loss_ref[...] = jnp.full((1, 1), loss, dtype=jnp.float32)


def _tc_argmin(x_cf, emb):
    return pl.pallas_call(
        _argmin_body,
        grid=(_B,),
        in_specs=[
            pl.BlockSpec((1, _C, _HW), lambda b: (b, 0, 0)),
            pl.BlockSpec((_K, _C), lambda b: (0, 0)),
        ],
        out_specs=[
            pl.BlockSpec((1, 1, _HW), lambda b: (b, 0, 0)),
            pl.BlockSpec((1, 1), lambda b: (0, 0)),
        ],
        out_shape=[
            jax.ShapeDtypeStruct((_B, 1, _HW), jnp.int32),
            jax.ShapeDtypeStruct((1, 1), jnp.float32),
        ],
        scratch_shapes=[
            pltpu.SMEM((1,), jnp.float32),
            pltpu.VMEM((_K, _C), jnp.float32),
            pltpu.VMEM((_K, 1), jnp.float32),
        ],
    )(x_cf, emb)


def _sc_body(idx_hbm, emb_hbm, zeros_hbm, xq_hbm, counts_hbm,
             idx_v, rows_v, ones_v, hist_sh, sem):
    c = lax.axis_index("c")
    s = lax.axis_index("s")
    wid = s * 2 + c
    base = wid * _TPW
    pltpu.sync_copy(idx_hbm.at[pl.ds(base, _HALF)], idx_v.at[0])
    pltpu.sync_copy(idx_hbm.at[pl.ds(base + _HALF, _HALF)], idx_v.at[1])
    cp0 = pltpu.async_copy(emb_hbm.at[idx_v.at[0]], rows_v.at[pl.ds(0, _HALF)], sem)
    cp1 = pltpu.async_copy(emb_hbm.at[idx_v.at[1]], rows_v.at[pl.ds(_HALF, _HALF)], sem)
    for i in range(_TPW // 16):
        ones_v[pl.ds(i * 16, 16)] = jnp.full((16,), 1.0, dtype=jnp.float32)

    @pl.when(s == 0)
    def _zero_hist():
        pltpu.sync_copy(zeros_hbm, hist_sh)

    plsc.subcore_barrier()
    pltpu.sync_copy(ones_v.at[pl.ds(0, _HALF)], hist_sh.at[idx_v.at[0]], add=True)
    pltpu.sync_copy(ones_v.at[pl.ds(_HALF, _HALF)], hist_sh.at[idx_v.at[1]], add=True)
    plsc.subcore_barrier()

    @pl.when(s == 0)
    def _write_counts():
        pltpu.sync_copy(hist_sh, counts_hbm.at[c])

    cp0.wait()
    cp1.wait()
    pltpu.sync_copy(rows_v, xq_hbm.at[pl.ds(base, _TPW)])


def _sc_gather_hist(idxs, emb, zeros):
    fn = pl.kernel(
        _sc_body,
        out_type=[
            jax.ShapeDtypeStruct((_N, _C), jnp.float32),
            jax.ShapeDtypeStruct((2, _K), jnp.float32),
        ],
        mesh=plsc.VectorSubcoreMesh(core_axis_name="c", subcore_axis_name="s"),
        scratch_types=[
            pltpu.VMEM((2, _HALF), jnp.int32),
            pltpu.VMEM((_TPW, _C), jnp.float32),
            pltpu.VMEM((_TPW,), jnp.float32),
            pltpu.VMEM_SHARED((_K,), jnp.float32),
            pltpu.SemaphoreType.DMA,
        ],
    )
    return fn(idxs, emb, zeros)


def _finish_body(xq_ref, cnt_ref, out_ref, perp_ref):
    b = pl.program_id(0)
    out_ref[0] = xq_ref[0].T

    @pl.when(b == _B - 1)
    def _tail():
        cnt = cnt_ref[0:1, :] + cnt_ref[1:2, :]
        p = cnt / jnp.float32(_N)
        ent = -jnp.sum(p * jnp.log(p + 1e-10))
        perp_ref[...] = jnp.full((1, 1), jnp.exp(ent), dtype=jnp.float32)


def _tc_finish(xq3, counts2):
    return pl.pallas_call(
        _finish_body,
        grid=(_B,),
        in_specs=[
            pl.BlockSpec((1, _HW, _C), lambda b: (b, 0, 0)),
            pl.BlockSpec((2, _K), lambda b: (0, 0)),
        ],
        out_specs=[
            pl.BlockSpec((1, _C, _HW), lambda b: (b, 0, 0)),
            pl.BlockSpec((1, 1), lambda b: (0, 0)),
        ],
        out_shape=[
            jax.ShapeDtypeStruct((_B, _C, _HW), jnp.float32),
            jax.ShapeDtypeStruct((1, 1), jnp.float32),
        ],
    )(xq3, counts2)


def kernel(x_in, emb_weight):
    x_cf = x_in.reshape(_B, _C, _HW)
    idxs3, loss = _tc_argmin(x_cf, emb_weight)
    idxs = idxs3.reshape(_N)
    zeros = jnp.zeros((_K,), jnp.float32)
    xq_flat, counts2 = _sc_gather_hist(idxs, emb_weight, zeros)
    xq3 = xq_flat.reshape(_B, _HW, _C)
    xq_out, perp = _tc_finish(xq3, counts2)
    return (
        xq_out.reshape(_B, _C, _H, _W),
        idxs.reshape(_B, _H, _W),
        loss[0, 0],
        perp[0, 0],
    )


# speedup vs baseline: 1.7780x; 1.0395x over previous
import jax
import jax.numpy as jnp
from jax import lax
from jax.experimental import pallas as pl
from jax.experimental.pallas import tpu as pltpu
from jax.experimental.pallas import tpu_sc as plsc

_K = 8192
_C = 256
_B = 8
_H = 24
_W = 24
_HW = _H * _W
_N = _B * _HW
_KCHUNK = 1024
_BETA = 0.25

_NW = 32
_TPW = _N // _NW
_HALF = _TPW // 2


def _argmin_body(x_ref, w_ref, idx_ref, loss_ref, acc_ref, wm2_ref, w2_ref):
    b = pl.program_id(0)

    @pl.when(b == 0)
    def _prep():
        w = w_ref[...]
        wm2_ref[...] = w * (-2.0)
        w2_ref[...] = jnp.sum(w * w, axis=1, keepdims=True)

    x_cb = x_ref[0]
    x2 = jnp.sum(x_cb * x_cb, axis=0)
    best8 = jnp.full((8, _HW), jnp.inf, dtype=jnp.float32)
    besti8 = jnp.zeros((8, _HW), dtype=jnp.int32)
    for k in range(_K // _KCHUNK):
        wv = wm2_ref[pl.ds(k * _KCHUNK, _KCHUNK), :]
        w2c = w2_ref[pl.ds(k * _KCHUNK, _KCHUNK), :]
        mm2 = lax.dot_general(wv, x_cb, (((1,), (0,)), ((), ())))
        d2 = (x2[None, :] + w2c) + mm2
        for i in range(_KCHUNK // 8):
            d2s = lax.slice(d2, (i * 8, 0), (i * 8 + 8, _HW))
            m = d2s < best8
            best8 = jnp.where(m, d2s, best8)
            besti8 = jnp.where(m, jnp.int32(k * (_KCHUNK // 8) + i), besti8)
    best_v = jnp.min(best8, axis=0)
    s_iota = lax.broadcasted_iota(jnp.int32, (8, _HW), 0)
    gk = besti8 * 8 + s_iota
    cand = jnp.where(best8 == best_v[None, :], gk, jnp.int32(_K))
    idx_ref[0, 0] = jnp.min(cand, axis=0)
    prev = jnp.where(b == 0, 0.0, acc_ref[0])
    acc_ref[0] = prev + jnp.sum(best_v)

    @pl.when(b == _B - 1)
    def _tail():
        loss = (1.0 + _BETA) * acc_ref[0] / jnp.float32(_N * _C)
        loss_ref[...] = jnp.full((1, 1), loss, dtype=jnp.float32)


def _tc_argmin(x_cf, emb):
    return pl.pallas_call(
        _argmin_body,
        grid=(_B,),
        in_specs=[
            pl.BlockSpec((1, _C, _HW), lambda b: (b, 0, 0)),
            pl.BlockSpec((_K, _C), lambda b: (0, 0)),
        ],
        out_specs=[
            pl.BlockSpec((1, 1, _HW), lambda b: (b, 0, 0)),
            pl.BlockSpec((1, 1), lambda b: (0, 0)),
        ],
        out_shape=[
            jax.ShapeDtypeStruct((_B, 1, _HW), jnp.int32),
            jax.ShapeDtypeStruct((1, 1), jnp.float32),
        ],
        scratch_shapes=[
            pltpu.SMEM((1,), jnp.float32),
            pltpu.VMEM((_K, _C), jnp.float32),
            pltpu.VMEM((_K, 1), jnp.float32),
        ],
    )(x_cf, emb)


def _sc_body(idx_hbm, emb_hbm, zeros_hbm, xq_hbm, counts_hbm,
             idx_v, rows_v, ones_v, hist_sh, sem):
    c = lax.axis_index("c")
    s = lax.axis_index("s")
    wid = s * 2 + c
    base = wid * _TPW
    pltpu.sync_copy(idx_hbm.at[pl.ds(base, _HALF)], idx_v.at[0])
    pltpu.sync_copy(idx_hbm.at[pl.ds(base + _HALF, _HALF)], idx_v.at[1])
    cp0 = pltpu.async_copy(emb_hbm.at[idx_v.at[0]], rows_v.at[pl.ds(0, _HALF)], sem)
    cp1 = pltpu.async_copy(emb_hbm.at[idx_v.at[1]], rows_v.at[pl.ds(_HALF, _HALF)], sem)
    for i in range(_TPW // 16):
        ones_v[pl.ds(i * 16, 16)] = jnp.full((16,), 1.0, dtype=jnp.float32)

    @pl.when(s == 0)
    def _zero_hist():
        pltpu.sync_copy(zeros_hbm, hist_sh)

    plsc.subcore_barrier()
    pltpu.sync_copy(ones_v.at[pl.ds(0, _HALF)], hist_sh.at[idx_v.at[0]], add=True)
    pltpu.sync_copy(ones_v.at[pl.ds(_HALF, _HALF)], hist_sh.at[idx_v.at[1]], add=True)
    plsc.subcore_barrier()

    @pl.when(s == 0)
    def _write_counts():
        pltpu.sync_copy(hist_sh, counts_hbm.at[c])

    cp0.wait()
    cp1.wait()
    pltpu.sync_copy(rows_v, xq_hbm.at[pl.ds(base, _TPW)])


def _sc_gather_hist(idxs, emb, zeros):
    fn = pl.kernel(
        _sc_body,
        out_type=[
            jax.ShapeDtypeStruct((_N, _C), jnp.float32),
            jax.ShapeDtypeStruct((2, _K), jnp.float32),
        ],
        mesh=plsc.VectorSubcoreMesh(core_axis_name="c", subcore_axis_name="s"),
        scratch_types=[
            pltpu.VMEM((2, _HALF), jnp.int32),
            pltpu.VMEM((_TPW, _C), jnp.float32),
            pltpu.VMEM((_TPW,), jnp.float32),
            pltpu.VMEM_SHARED((_K,), jnp.float32),
            pltpu.SemaphoreType.DMA,
        ],
    )
    return fn(idxs, emb, zeros)


def _finish_body(xq_ref, cnt_ref, out_ref, perp_ref):
    b = pl.program_id(0)
    for j in range(4):
        out_ref[j] = xq_ref[j].T

    @pl.when(b == 1)
    def _tail():
        cnt = cnt_ref[0:1, :] + cnt_ref[1:2, :]
        p = cnt / jnp.float32(_N)
        ent = -jnp.sum(p * jnp.log(p + 1e-10))
        perp_ref[...] = jnp.full((1, 1), jnp.exp(ent), dtype=jnp.float32)


def _tc_finish(xq3, counts2):
    return pl.pallas_call(
        _finish_body,
        grid=(2,),
        in_specs=[
            pl.BlockSpec((4, _HW, _C), lambda b: (b, 0, 0)),
            pl.BlockSpec((2, _K), lambda b: (0, 0)),
        ],
        out_specs=[
            pl.BlockSpec((4, _C, _HW), lambda b: (b, 0, 0)),
            pl.BlockSpec((1, 1), lambda b: (0, 0)),
        ],
        out_shape=[
            jax.ShapeDtypeStruct((_B, _C, _HW), jnp.float32),
            jax.ShapeDtypeStruct((1, 1), jnp.float32),
        ],
    )(xq3, counts2)


def kernel(x_in, emb_weight):
    x_cf = x_in.reshape(_B, _C, _HW)
    idxs3, loss = _tc_argmin(x_cf, emb_weight)
    idxs = idxs3.reshape(_N)
    zeros = jnp.zeros((_K,), jnp.float32)
    xq_flat, counts2 = _sc_gather_hist(idxs, emb_weight, zeros)
    xq3 = xq_flat.reshape(_B, _HW, _C)
    xq_out, perp = _tc_finish(xq3, counts2)
    return (
        xq_out.reshape(_B, _C, _H, _W),
        idxs.reshape(_B, _H, _W),
        loss[0, 0],
        perp[0, 0],
    )


# speedup vs baseline: 1.7827x; 1.0026x over previous
import jax
import jax.numpy as jnp
from jax import lax
from jax.experimental import pallas as pl
from jax.experimental.pallas import tpu as pltpu
from jax.experimental.pallas import tpu_sc as plsc

_K = 8192
_C = 256
_B = 8
_H = 24
_W = 24
_HW = _H * _W
_N = _B * _HW
_KCHUNK = 2048
_BETA = 0.25

_NW = 32
_TPW = _N // _NW
_HALF = _TPW // 2


def _argmin_body(x_ref, w_ref, idx_ref, loss_ref, acc_ref, wm2_ref, w2_ref):
    b = pl.program_id(0)

    @pl.when(b == 0)
    def _prep():
        w = w_ref[...]
        wm2_ref[...] = w * (-2.0)
        w2_ref[...] = jnp.sum(w * w, axis=1, keepdims=True)

    x_cb = x_ref[0]
    x2 = jnp.sum(x_cb * x_cb, axis=0)
    best8 = jnp.full((8, _HW), jnp.inf, dtype=jnp.float32)
    besti8 = jnp.zeros((8, _HW), dtype=jnp.int32)
    for k in range(_K // _KCHUNK):
        wv = wm2_ref[pl.ds(k * _KCHUNK, _KCHUNK), :]
        w2c = w2_ref[pl.ds(k * _KCHUNK, _KCHUNK), :]
        mm2 = lax.dot_general(wv, x_cb, (((1,), (0,)), ((), ())))
        d2 = (x2[None, :] + w2c) + mm2
        for i in range(_KCHUNK // 8):
            d2s = lax.slice(d2, (i * 8, 0), (i * 8 + 8, _HW))
            m = d2s < best8
            best8 = jnp.where(m, d2s, best8)
            besti8 = jnp.where(m, jnp.int32(k * (_KCHUNK // 8) + i), besti8)
    best_v = jnp.min(best8, axis=0)
    s_iota = lax.broadcasted_iota(jnp.int32, (8, _HW), 0)
    gk = besti8 * 8 + s_iota
    cand = jnp.where(best8 == best_v[None, :], gk, jnp.int32(_K))
    idx_ref[0, 0] = jnp.min(cand, axis=0)
    prev = jnp.where(b == 0, 0.0, acc_ref[0])
    acc_ref[0] = prev + jnp.sum(best_v)

    @pl.when(b == _B - 1)
    def _tail():
        loss = (1.0 + _BETA) * acc_ref[0] / jnp.float32(_N * _C)
        loss_ref[...] = jnp.full((1, 1), loss, dtype=jnp.float32)


def _tc_argmin(x_cf, emb):
    return pl.pallas_call(
        _argmin_body,
        grid=(_B,),
        in_specs=[
            pl.BlockSpec((1, _C, _HW), lambda b: (b, 0, 0)),
            pl.BlockSpec((_K, _C), lambda b: (0, 0)),
        ],
        out_specs=[
            pl.BlockSpec((1, 1, _HW), lambda b: (b, 0, 0)),
            pl.BlockSpec((1, 1), lambda b: (0, 0)),
        ],
        out_shape=[
            jax.ShapeDtypeStruct((_B, 1, _HW), jnp.int32),
            jax.ShapeDtypeStruct((1, 1), jnp.float32),
        ],
        scratch_shapes=[
            pltpu.SMEM((1,), jnp.float32),
            pltpu.VMEM((_K, _C), jnp.float32),
            pltpu.VMEM((_K, 1), jnp.float32),
        ],
    )(x_cf, emb)


def _sc_body(idx_hbm, emb_hbm, zeros_hbm, xq_hbm, counts_hbm,
             idx_v, rows_v, ones_v, hist_sh, sem):
    c = lax.axis_index("c")
    s = lax.axis_index("s")
    wid = s * 2 + c
    base = wid * _TPW
    pltpu.sync_copy(idx_hbm.at[pl.ds(base, _HALF)], idx_v.at[0])
    pltpu.sync_copy(idx_hbm.at[pl.ds(base + _HALF, _HALF)], idx_v.at[1])
    cp0 = pltpu.async_copy(emb_hbm.at[idx_v.at[0]], rows_v.at[pl.ds(0, _HALF)], sem)
    cp1 = pltpu.async_copy(emb_hbm.at[idx_v.at[1]], rows_v.at[pl.ds(_HALF, _HALF)], sem)
    for i in range(_TPW // 16):
        ones_v[pl.ds(i * 16, 16)] = jnp.full((16,), 1.0, dtype=jnp.float32)

    @pl.when(s == 0)
    def _zero_hist():
        pltpu.sync_copy(zeros_hbm, hist_sh)

    plsc.subcore_barrier()
    pltpu.sync_copy(ones_v.at[pl.ds(0, _HALF)], hist_sh.at[idx_v.at[0]], add=True)
    pltpu.sync_copy(ones_v.at[pl.ds(_HALF, _HALF)], hist_sh.at[idx_v.at[1]], add=True)
    plsc.subcore_barrier()

    @pl.when(s == 0)
    def _write_counts():
        pltpu.sync_copy(hist_sh, counts_hbm.at[c])

    cp0.wait()
    cp1.wait()
    pltpu.sync_copy(rows_v, xq_hbm.at[pl.ds(base, _TPW)])


def _sc_gather_hist(idxs, emb, zeros):
    fn = pl.kernel(
        _sc_body,
        out_type=[
            jax.ShapeDtypeStruct((_N, _C), jnp.float32),
            jax.ShapeDtypeStruct((2, _K), jnp.float32),
        ],
        mesh=plsc.VectorSubcoreMesh(core_axis_name="c", subcore_axis_name="s"),
        scratch_types=[
            pltpu.VMEM((2, _HALF), jnp.int32),
            pltpu.VMEM((_TPW, _C), jnp.float32),
            pltpu.VMEM((_TPW,), jnp.float32),
            pltpu.VMEM_SHARED((_K,), jnp.float32),
            pltpu.SemaphoreType.DMA,
        ],
    )
    return fn(idxs, emb, zeros)


def _finish_body(xq_ref, cnt_ref, out_ref, perp_ref):
    b = pl.program_id(0)
    for j in range(4):
        out_ref[j] = xq_ref[j].T

    @pl.when(b == 1)
    def _tail():
        cnt = cnt_ref[0:1, :] + cnt_ref[1:2, :]
        p = cnt / jnp.float32(_N)
        ent = -jnp.sum(p * jnp.log(p + 1e-10))
        perp_ref[...] = jnp.full((1, 1), jnp.exp(ent), dtype=jnp.float32)


def _tc_finish(xq3, counts2):
    return pl.pallas_call(
        _finish_body,
        grid=(2,),
        in_specs=[
            pl.BlockSpec((4, _HW, _C), lambda b: (b, 0, 0)),
            pl.BlockSpec((2, _K), lambda b: (0, 0)),
        ],
        out_specs=[
            pl.BlockSpec((4, _C, _HW), lambda b: (b, 0, 0)),
            pl.BlockSpec((1, 1), lambda b: (0, 0)),
        ],
        out_shape=[
            jax.ShapeDtypeStruct((_B, _C, _HW), jnp.float32),
            jax.ShapeDtypeStruct((1, 1), jnp.float32),
        ],
    )(xq3, counts2)


def kernel(x_in, emb_weight):
    x_cf = x_in.reshape(_B, _C, _HW)
    idxs3, loss = _tc_argmin(x_cf, emb_weight)
    idxs = idxs3.reshape(_N)
    zeros = jnp.zeros((_K,), jnp.float32)
    xq_flat, counts2 = _sc_gather_hist(idxs, emb_weight, zeros)
    xq3 = xq_flat.reshape(_B, _HW, _C)
    xq_out, perp = _tc_finish(xq3, counts2)
    return (
        xq_out.reshape(_B, _C, _H, _W),
        idxs.reshape(_B, _H, _W),
        loss[0, 0],
        perp[0, 0],
    )


# speedup vs baseline: 1.8063x; 1.0133x over previous
import jax
import jax.numpy as jnp
from jax import lax
from jax.experimental import pallas as pl
from jax.experimental.pallas import tpu as pltpu
from jax.experimental.pallas import tpu_sc as plsc

_K = 8192
_C = 256
_B = 8
_H = 24
_W = 24
_HW = _H * _W
_N = _B * _HW
_KCHUNK = 2048
_BETA = 0.25

_NW = 32
_TPW = _N // _NW
_HALF = _TPW // 2


def _argmin_body(x_ref, w_ref, idx_ref, loss_ref, acc_ref, wm2_ref, w2_ref):
    b = pl.program_id(0)

    @pl.when(b == 0)
    def _prep():
        w = w_ref[...]
        wm2_ref[...] = w * (-2.0)
        w2_ref[...] = jnp.sum(w * w, axis=1, keepdims=True)

    x_cb = x_ref[0]
    x2 = jnp.sum(x_cb * x_cb, axis=0)
    best8 = jnp.full((8, _HW), jnp.inf, dtype=jnp.float32)
    besti8 = jnp.zeros((8, _HW), dtype=jnp.int32)
    for k in range(_K // _KCHUNK):
        wv = wm2_ref[pl.ds(k * _KCHUNK, _KCHUNK), :]
        w2c = w2_ref[pl.ds(k * _KCHUNK, _KCHUNK), :]
        mm2 = lax.dot_general(wv, x_cb, (((1,), (0,)), ((), ())))
        d2 = (x2[None, :] + w2c) + mm2
        for i in range(_KCHUNK // 8):
            d2s = lax.slice(d2, (i * 8, 0), (i * 8 + 8, _HW))
            m = d2s < best8
            best8 = jnp.where(m, d2s, best8)
            besti8 = jnp.where(m, jnp.int32(k * (_KCHUNK // 8) + i), besti8)
    best_v = jnp.min(best8, axis=0)
    s_iota = lax.broadcasted_iota(jnp.int32, (8, _HW), 0)
    gk = besti8 * 8 + s_iota
    cand = jnp.where(best8 == best_v[None, :], gk, jnp.int32(_K))
    idx_ref[0, 0] = jnp.min(cand, axis=0)
    prev = jnp.where(b == 0, 0.0, acc_ref[0])
    acc_ref[0] = prev + jnp.sum(best_v)

    @pl.when(b == _B - 1)
    def _tail():
        loss = (1.0 + _BETA) * acc_ref[0] / jnp.float32(_N * _C)
        loss_ref[...] = jnp.full((1, 1), loss, dtype=jnp.float32)


def _tc_argmin(x_cf, emb):
    return pl.pallas_call(
        _argmin_body,
        grid=(_B,),
        in_specs=[
            pl.BlockSpec((1, _C, _HW), lambda b: (b, 0, 0)),
            pl.BlockSpec((_K, _C), lambda b: (0, 0)),
        ],
        out_specs=[
            pl.BlockSpec((1, 1, _HW), lambda b: (b, 0, 0)),
            pl.BlockSpec((1, 1), lambda b: (0, 0)),
        ],
        out_shape=[
            jax.ShapeDtypeStruct((_B, 1, _HW), jnp.int32),
            jax.ShapeDtypeStruct((1, 1), jnp.float32),
        ],
        scratch_shapes=[
            pltpu.SMEM((1,), jnp.float32),
            pltpu.VMEM((_K, _C), jnp.float32),
            pltpu.VMEM((_K, 1), jnp.float32),
        ],
    )(x_cf, emb)


def _sc_body(idx_hbm, emb_hbm, xq_hbm, counts_hbm,
             idx_v, rows_v, ones_v, zbuf_v, hist_sh, sem, sem2):
    c = lax.axis_index("c")
    s = lax.axis_index("s")
    wid = s * 2 + c
    base = wid * _TPW
    cpi0 = pltpu.async_copy(idx_hbm.at[pl.ds(base, _HALF)], idx_v.at[0], sem2)
    cpi1 = pltpu.async_copy(idx_hbm.at[pl.ds(base + _HALF, _HALF)], idx_v.at[1], sem2)
    for i in range(_TPW // 16):
        ones_v[pl.ds(i * 16, 16)] = jnp.full((16,), 1.0, dtype=jnp.float32)
    for i in range(512 // 16):
        zbuf_v[pl.ds(i * 16, 16)] = jnp.zeros((16,), dtype=jnp.float32)
    cpi0.wait()
    cpi1.wait()
    cp0 = pltpu.async_copy(emb_hbm.at[idx_v.at[0]], rows_v.at[pl.ds(0, _HALF)], sem)
    cp1 = pltpu.async_copy(emb_hbm.at[idx_v.at[1]], rows_v.at[pl.ds(_HALF, _HALF)], sem)
    pltpu.sync_copy(zbuf_v, hist_sh.at[pl.ds(s * (_K // 16), _K // 16)])
    cp0.wait()
    cp1.wait()
    wb = pltpu.async_copy(rows_v, xq_hbm.at[pl.ds(base, _TPW)], sem2)
    plsc.subcore_barrier()
    pltpu.sync_copy(ones_v.at[pl.ds(0, _HALF)], hist_sh.at[idx_v.at[0]], add=True)
    pltpu.sync_copy(ones_v.at[pl.ds(_HALF, _HALF)], hist_sh.at[idx_v.at[1]], add=True)
    plsc.subcore_barrier()

    @pl.when(s == 0)
    def _write_counts():
        pltpu.sync_copy(hist_sh, counts_hbm.at[c])

    wb.wait()


def _sc_gather_hist(idxs, emb):
    fn = pl.kernel(
        _sc_body,
        out_type=[
            jax.ShapeDtypeStruct((_N, _C), jnp.float32),
            jax.ShapeDtypeStruct((2, _K), jnp.float32),
        ],
        mesh=plsc.VectorSubcoreMesh(core_axis_name="c", subcore_axis_name="s"),
        scratch_types=[
            pltpu.VMEM((2, _HALF), jnp.int32),
            pltpu.VMEM((_TPW, _C), jnp.float32),
            pltpu.VMEM((_TPW,), jnp.float32),
            pltpu.VMEM((_K // 16,), jnp.float32),
            pltpu.VMEM_SHARED((_K,), jnp.float32),
            pltpu.SemaphoreType.DMA,
            pltpu.SemaphoreType.DMA,
        ],
    )
    return fn(idxs, emb)


def _finish_body(xq_ref, cnt_ref, out_ref, perp_ref):
    b = pl.program_id(0)
    for j in range(4):
        out_ref[j] = xq_ref[j].T

    @pl.when(b == 1)
    def _tail():
        cnt = cnt_ref[0:1, :] + cnt_ref[1:2, :]
        p = cnt / jnp.float32(_N)
        ent = -jnp.sum(p * jnp.log(p + 1e-10))
        perp_ref[...] = jnp.full((1, 1), jnp.exp(ent), dtype=jnp.float32)


def _tc_finish(xq3, counts2):
    return pl.pallas_call(
        _finish_body,
        grid=(2,),
        in_specs=[
            pl.BlockSpec((4, _HW, _C), lambda b: (b, 0, 0)),
            pl.BlockSpec((2, _K), lambda b: (0, 0)),
        ],
        out_specs=[
            pl.BlockSpec((4, _C, _HW), lambda b: (b, 0, 0)),
            pl.BlockSpec((1, 1), lambda b: (0, 0)),
        ],
        out_shape=[
            jax.ShapeDtypeStruct((_B, _C, _HW), jnp.float32),
            jax.ShapeDtypeStruct((1, 1), jnp.float32),
        ],
    )(xq3, counts2)


def kernel(x_in, emb_weight):
    x_cf = x_in.reshape(_B, _C, _HW)
    idxs3, loss = _tc_argmin(x_cf, emb_weight)
    idxs = idxs3.reshape(_N)
    xq_flat, counts2 = _sc_gather_hist(idxs, emb_weight)
    xq3 = xq_flat.reshape(_B, _HW, _C)
    xq_out, perp = _tc_finish(xq3, counts2)
    return (
        xq_out.reshape(_B, _C, _H, _W),
        idxs.reshape(_B, _H, _W),
        loss[0, 0],
        perp[0, 0],
    )
